# baseline (speedup 1.0000x reference)
import dataclasses
import functools

import jax
import jax.numpy as jnp
from jax import lax
from jax.experimental import pallas as pl
from jax.experimental.pallas import tpu as pltpu
from jax.experimental.pallas import tpu_sc as plsc

N = 10000
E = 320000
NC = 2
NS = 16
NW = NC * NS
ET = E // NW
C = 80
NCHUNK = ET // C
L = 16
NP = 10112
SP = NP // NS

_mesh = plsc.VectorSubcoreMesh(core_axis_name="c", subcore_axis_name="s")

_sc_params = pltpu.CompilerParams()
if "needs_layout_passes" in pltpu.CompilerParams.__dataclass_fields__:
    _sc_params = dataclasses.replace(
        _sc_params, needs_layout_passes=False, use_tc_tiling_on_sc=False)


def _att_pass(xn, src3, dst3, zrows, D):
    @functools.partial(
        pl.kernel,
        out_type=(
            jax.ShapeDtypeStruct((NW, NCHUNK, C), jnp.float32),
            jax.ShapeDtypeStruct((NC, NP, L), jnp.float32),
        ),
        mesh=_mesh,
        compiler_params=_sc_params,
        scratch_types=[
            pltpu.VMEM((NCHUNK, C), jnp.int32),
            pltpu.VMEM((NCHUNK, C), jnp.int32),
            pltpu.VMEM((C, D), jnp.float32),
            pltpu.VMEM((C, D), jnp.float32),
            pltpu.VMEM((C, D), jnp.float32),
            pltpu.VMEM((C, D), jnp.float32),
            pltpu.VMEM((NCHUNK, C), jnp.float32),
            pltpu.VMEM((C, L), jnp.float32),
            pltpu.VMEM((C, L), jnp.float32),
            pltpu.VMEM_SHARED((NP, L), jnp.float32),
            pltpu.SemaphoreType.DMA,
            pltpu.SemaphoreType.DMA,
            pltpu.SemaphoreType.DMA,
            pltpu.SemaphoreType.DMA,
            pltpu.SemaphoreType.DMA,
            pltpu.SemaphoreType.DMA,
        ],
    )
    def att(xn_hbm, src_hbm, dst_hbm, z_hbm, ew_hbm, degp_hbm,
            idx_sa, idx_da, a0, a1, b0, b1, ew_all, dr0, dr1, deg_sh,
            sa0, sa1, sb0, sb1, sd0, sd1):
        cid = lax.axis_index("c")
        sid = lax.axis_index("s")
        wid = sid * NC + cid
        a_bufs, b_bufs = (a0, a1), (b0, b1)
        drows = (dr0, dr1)
        sas, sbs, sds = (sa0, sa1), (sb0, sb1), (sd0, sd1)

        pltpu.sync_copy(z_hbm.at[pl.ds(0, SP)], deg_sh.at[pl.ds(sid * SP, SP)])
        pltpu.sync_copy(z_hbm.at[pl.ds(0, C)], dr0)
        pltpu.sync_copy(z_hbm.at[pl.ds(0, C)], dr1)
        pltpu.sync_copy(src_hbm.at[wid], idx_sa)
        pltpu.sync_copy(dst_hbm.at[wid], idx_da)
        plsc.subcore_barrier()

        lane_iota = lax.iota(jnp.int32, L)
        zeros_i = jnp.zeros((L,), jnp.int32)

        def issue(i, p):
            pltpu.async_copy(xn_hbm.at[idx_sa.at[i]], a_bufs[p], sas[p])
            pltpu.async_copy(xn_hbm.at[idx_da.at[i]], b_bufs[p], sbs[p])

        def wait_dscat(p):
            pltpu.make_async_copy(
                drows[p], deg_sh.at[idx_da.at[0]], sds[p]).wait()

        def step(i, p, issue_next, wait_d):
            if issue_next:
                issue(i + 1, 1 - p)
            pltpu.make_async_copy(
                xn_hbm.at[idx_sa.at[i]], a_bufs[p], sas[p]).wait()
            pltpu.make_async_copy(
                xn_hbm.at[idx_da.at[i]], b_bufs[p], sbs[p]).wait()
            if wait_d:
                wait_dscat(p)
            a_buf, b_buf, drow = a_bufs[p], b_bufs[p], drows[p]

            @pl.loop(0, C // L)
            def _(g):
                sims = jnp.zeros((L,), jnp.float32)
                for e in range(L):
                    row = g * L + e
                    acc = a_buf[row, pl.ds(0, L)] * b_buf[row, pl.ds(0, L)]
                    for k in range(1, D // L):
                        acc += (a_buf[row, pl.ds(k * L, L)]
                                * b_buf[row, pl.ds(k * L, L)])
                    sims = jnp.where(lane_iota == e,
                                     jnp.broadcast_to(jnp.sum(acc), (L,)),
                                     sims)
                sims = jnp.where(sims < 0.1, 0.0, sims)
                ew_all[i, pl.ds(g * L, L)] = sims
                plsc.store_scatter(drow, [g * L + lane_iota, zeros_i], sims)

            pltpu.async_copy(drow, deg_sh.at[idx_da.at[i]], sds[p], add=True)

        issue(0, 0)
        step(0, 0, True, False)
        step(1, 1, True, False)

        @pl.loop(0, (NCHUNK - 3) // 2)
        def _(j):
            step(2 * j + 2, 0, True, True)
            step(2 * j + 3, 1, True, True)

        step(NCHUNK - 1, 0, False, True)
        wait_dscat(1)
        wait_dscat(0)

        pltpu.sync_copy(ew_all, ew_hbm.at[wid])
        plsc.subcore_barrier()
        pltpu.sync_copy(deg_sh.at[pl.ds(sid * SP, SP)],
                        degp_hbm.at[cid, pl.ds(sid * SP, SP)])

    return att(xn, src3, dst3, zrows)


def _msg_pass(h, src3, dst3, ew3, dinv, zrows, Dm):
    @functools.partial(
        pl.kernel,
        out_type=jax.ShapeDtypeStruct((NC, NP, Dm), jnp.float32),
        mesh=_mesh,
        compiler_params=_sc_params,
        scratch_types=[
            pltpu.VMEM((NP,), jnp.float32),
            pltpu.VMEM((NCHUNK, C), jnp.int32),
            pltpu.VMEM((NCHUNK, C), jnp.int32),
            pltpu.VMEM((NCHUNK, C), jnp.float32),
            pltpu.VMEM((C, Dm), jnp.float32),
            pltpu.VMEM((C, Dm), jnp.float32),
            pltpu.VMEM_SHARED((NP, Dm), jnp.float32),
            pltpu.SemaphoreType.DMA,
            pltpu.SemaphoreType.DMA,
            pltpu.SemaphoreType.DMA,
            pltpu.SemaphoreType.DMA,
        ],
    )
    def msg(h_hbm, src_hbm, dst_hbm, ew_hbm, dinv_hbm, z_hbm, mp_hbm,
            dinv_v, idx_sa, idx_da, ew_all, r0, r1, acc_sh,
            s0, s1, sc0, sc1):
        cid = lax.axis_index("c")
        sid = lax.axis_index("s")
        wid = sid * NC + cid
        rows_bufs, sems, scs = (r0, r1), (s0, s1), (sc0, sc1)

        pltpu.sync_copy(z_hbm, acc_sh.at[pl.ds(sid * SP, SP)])
        pltpu.sync_copy(dinv_hbm, dinv_v)
        pltpu.sync_copy(src_hbm.at[wid], idx_sa)
        pltpu.sync_copy(dst_hbm.at[wid], idx_da)
        pltpu.sync_copy(ew_hbm.at[wid], ew_all)
        plsc.subcore_barrier()

        def issue(i, p):
            pltpu.async_copy(h_hbm.at[idx_sa.at[i]], rows_bufs[p], sems[p])

        def wait_scat(p):
            pltpu.make_async_copy(
                rows_bufs[p], acc_sh.at[idx_da.at[0]], scs[p]).wait()

        def step(i, p, issue_next, wait_sc):
            if wait_sc:
                wait_scat(1 - p)
            if issue_next:
                issue(i + 1, 1 - p)
            pltpu.make_async_copy(
                h_hbm.at[idx_sa.at[i]], rows_bufs[p], sems[p]).wait()
            rows = rows_bufs[p]

            @pl.loop(0, C // L)
            def _(g):
                isv = idx_sa[i, pl.ds(g * L, L)]
                idv = idx_da[i, pl.ds(g * L, L)]
                ds_ = plsc.load_gather(dinv_v, [isv])
                dd_ = plsc.load_gather(dinv_v, [idv])
                w = ds_ * ew_all[i, pl.ds(g * L, L)] * dd_
                for e in range(L):
                    row = g * L + e
                    wv = jnp.broadcast_to(w[e], (L,))
                    for k in range(Dm // L):
                        rows[row, pl.ds(k * L, L)] = (
                            rows[row, pl.ds(k * L, L)] * wv)

            pltpu.async_copy(rows, acc_sh.at[idx_da.at[i]], scs[p], add=True)

        issue(0, 0)
        step(0, 0, True, False)

        @pl.loop(0, (NCHUNK - 3) // 2)
        def _(j):
            step(2 * j + 1, 1, True, True)
            step(2 * j + 2, 0, True, True)

        step(NCHUNK - 2, 1, True, True)
        step(NCHUNK - 1, 0, False, True)
        wait_scat(0)

        plsc.subcore_barrier()
        pltpu.sync_copy(acc_sh.at[pl.ds(sid * SP, SP)],
                        mp_hbm.at[cid, pl.ds(sid * SP, SP)])

    return msg(h, src3, dst3, ew3, dinv, zrows)


def _tc_call(body, out_shape, *args):
    return pl.pallas_call(body, out_shape=out_shape)(*args)


def _prep1(x, W1):
    def body(x_ref, w_ref, xn_ref, h1_ref):
        xv = x_ref[...]
        s = jnp.sum(xv * xv, axis=1, keepdims=True)
        na = jnp.maximum(jnp.sqrt(s), 1e-8)
        xn_ref[...] = xv / na
        h1_ref[...] = jnp.dot(xv, w_ref[...],
                              preferred_element_type=jnp.float32)
    return _tc_call(
        body,
        (jax.ShapeDtypeStruct((N, x.shape[1]), jnp.float32),
         jax.ShapeDtypeStruct((N, W1.shape[1]), jnp.float32)),
        x, W1)


def _dinv_of(degp):
    def body(degp_ref, dinv_ref):
        deg = 1.0 + jnp.sum(degp_ref[...], axis=(0, 2), keepdims=True)
        dinv_ref[...] = lax.rsqrt(deg)
    return _tc_call(body, jax.ShapeDtypeStruct((1, NP, 1), jnp.float32), degp)


def _mid(mp, h1, dinv_col, b1_row, W2):
    def body(mp_ref, h1_ref, dc_ref, b_ref, w_ref, hn_ref, h2_ref):
        dc = dc_ref[...][:N]
        h = (mp_ref[0][:N] + mp_ref[1][:N]
             + dc * dc * h1_ref[...] + b_ref[...])
        h = jnp.maximum(h, 0.0)
        s = jnp.sum(h * h, axis=1, keepdims=True)
        na = jnp.maximum(jnp.sqrt(s), 1e-8)
        hn_ref[...] = h / na
        h2_ref[...] = jnp.dot(h, w_ref[...],
                              preferred_element_type=jnp.float32)
    return _tc_call(
        body,
        (jax.ShapeDtypeStruct((N, h1.shape[1]), jnp.float32),
         jax.ShapeDtypeStruct((N, W2.shape[1]), jnp.float32)),
        mp, h1, dinv_col, b1_row, W2)


def _final(mp, h2, dinv_col, b2_row):
    def body(mp_ref, h2_ref, dc_ref, b_ref, out_ref):
        dc = dc_ref[...][:N]
        out_ref[...] = (mp_ref[0][:N] + mp_ref[1][:N]
                        + dc * dc * h2_ref[...] + b_ref[...])
    return _tc_call(
        body, jax.ShapeDtypeStruct((N, h2.shape[1]), jnp.float32),
        mp, h2, dinv_col, b2_row)


def kernel(x, adj, W1, b1, W2, b2):
    src3 = adj[0].astype(jnp.int32).reshape(NW, NCHUNK, C)
    dst3 = adj[1].astype(jnp.int32).reshape(NW, NCHUNK, C)
    z16 = jnp.zeros((SP, L), jnp.float32)

    xn, h1 = _prep1(x, W1)
    ew1, degp1 = _att_pass(xn, src3, dst3, z16, x.shape[1])
    dinv1_3 = _dinv_of(degp1)
    mp1 = _msg_pass(h1, src3, dst3, ew1, dinv1_3.reshape(NP),
                    jnp.zeros((SP, h1.shape[1]), jnp.float32), h1.shape[1])
    hn, h2 = _mid(mp1, h1, dinv1_3.reshape(NP, 1), b1.reshape(1, -1), W2)
    ew2, degp2 = _att_pass(hn, src3, dst3, z16, hn.shape[1])
    dinv2_3 = _dinv_of(degp2)
    mp2 = _msg_pass(h2, src3, dst3, ew2, dinv2_3.reshape(NP),
                    jnp.zeros((SP, h2.shape[1]), jnp.float32), h2.shape[1])
    return _final(mp2, h2, dinv2_3.reshape(NP, 1), b2.reshape(1, -1))


# speedup vs baseline: 1.3183x; 1.3183x over previous
import dataclasses
import functools

import jax
import jax.numpy as jnp
from jax import lax
from jax.experimental import pallas as pl
from jax.experimental.pallas import tpu as pltpu
from jax.experimental.pallas import tpu_sc as plsc

N = 10000
E = 320000
NC = 2
NS = 16
NW = NC * NS
ET = E // NW
C = 80
NCHUNK = ET // C
L = 16
NP = 10112
SP = NP // NS

_mesh = plsc.VectorSubcoreMesh(core_axis_name="c", subcore_axis_name="s")

_sc_params = pltpu.CompilerParams()
if "needs_layout_passes" in pltpu.CompilerParams.__dataclass_fields__:
    _sc_params = dataclasses.replace(
        _sc_params, needs_layout_passes=False, use_tc_tiling_on_sc=False)


def _att_pass(xn, src3, dst3, zrows, D):
    @functools.partial(
        pl.kernel,
        out_type=(
            jax.ShapeDtypeStruct((NW, NCHUNK, C), jnp.float32),
            jax.ShapeDtypeStruct((NC, NP, L), jnp.float32),
        ),
        mesh=_mesh,
        compiler_params=_sc_params,
        scratch_types=[
            pltpu.VMEM((NCHUNK, C), jnp.int32),
            pltpu.VMEM((NCHUNK, C), jnp.int32),
            pltpu.VMEM((C, D), jnp.float32),
            pltpu.VMEM((C, D), jnp.float32),
            pltpu.VMEM((C, D), jnp.float32),
            pltpu.VMEM((C, D), jnp.float32),
            pltpu.VMEM((NCHUNK, C), jnp.float32),
            pltpu.VMEM((C, L), jnp.float32),
            pltpu.VMEM((C, L), jnp.float32),
            pltpu.VMEM_SHARED((NP, L), jnp.float32),
            pltpu.SemaphoreType.DMA,
            pltpu.SemaphoreType.DMA,
            pltpu.SemaphoreType.DMA,
            pltpu.SemaphoreType.DMA,
            pltpu.SemaphoreType.DMA,
            pltpu.SemaphoreType.DMA,
        ],
    )
    def att(xn_hbm, src_hbm, dst_hbm, z_hbm, ew_hbm, degp_hbm,
            idx_sa, idx_da, a0, a1, b0, b1, ew_all, dr0, dr1, deg_sh,
            sa0, sa1, sb0, sb1, sd0, sd1):
        cid = lax.axis_index("c")
        sid = lax.axis_index("s")
        wid = sid * NC + cid
        a_bufs, b_bufs = (a0, a1), (b0, b1)
        drows = (dr0, dr1)
        sas, sbs, sds = (sa0, sa1), (sb0, sb1), (sd0, sd1)

        pltpu.sync_copy(z_hbm.at[pl.ds(0, SP)], deg_sh.at[pl.ds(sid * SP, SP)])
        pltpu.sync_copy(z_hbm.at[pl.ds(0, C)], dr0)
        pltpu.sync_copy(z_hbm.at[pl.ds(0, C)], dr1)
        pltpu.sync_copy(src_hbm.at[wid], idx_sa)
        pltpu.sync_copy(dst_hbm.at[wid], idx_da)
        plsc.subcore_barrier()

        lane_iota = lax.iota(jnp.int32, L)
        zeros_i = jnp.zeros((L,), jnp.int32)

        def issue(i, p):
            pltpu.async_copy(xn_hbm.at[idx_sa.at[i]], a_bufs[p], sas[p])
            pltpu.async_copy(xn_hbm.at[idx_da.at[i]], b_bufs[p], sbs[p])

        def wait_dscat(p):
            pltpu.make_async_copy(
                drows[p], deg_sh.at[idx_da.at[0]], sds[p]).wait()

        def step(i, p, issue_next, wait_d):
            if issue_next:
                issue(i + 1, 1 - p)
            pltpu.make_async_copy(
                xn_hbm.at[idx_sa.at[i]], a_bufs[p], sas[p]).wait()
            pltpu.make_async_copy(
                xn_hbm.at[idx_da.at[i]], b_bufs[p], sbs[p]).wait()
            if wait_d:
                wait_dscat(p)
            a_buf, b_buf, drow = a_bufs[p], b_bufs[p], drows[p]

            @pl.loop(0, C // 4)
            def _(q):
                sims4 = jnp.zeros((L,), jnp.float32)
                for e in range(4):
                    row = q * 4 + e
                    acc = a_buf[row, pl.ds(0, L)] * b_buf[row, pl.ds(0, L)]
                    for k in range(1, D // L):
                        acc += (a_buf[row, pl.ds(k * L, L)]
                                * b_buf[row, pl.ds(k * L, L)])
                    sims4 = jnp.where(lane_iota == e,
                                      jnp.broadcast_to(jnp.sum(acc), (L,)),
                                      sims4)
                sims4 = jnp.where(sims4 < 0.1, 0.0, sims4)
                plsc.store_scatter(
                    ew_all,
                    [jnp.full((L,), i, jnp.int32), q * 4 + lane_iota],
                    sims4, mask=lane_iota < 4)

            @pl.loop(0, C // L)
            def _(g):
                plsc.store_scatter(drow, [g * L + lane_iota, zeros_i],
                                   ew_all[i, pl.ds(g * L, L)])

            pltpu.async_copy(drow, deg_sh.at[idx_da.at[i]], sds[p], add=True)

        issue(0, 0)
        step(0, 0, True, False)
        step(1, 1, True, False)

        @pl.loop(0, (NCHUNK - 3) // 2)
        def _(j):
            step(2 * j + 2, 0, True, True)
            step(2 * j + 3, 1, True, True)

        step(NCHUNK - 1, 0, False, True)
        wait_dscat(1)
        wait_dscat(0)

        pltpu.sync_copy(ew_all, ew_hbm.at[wid])
        plsc.subcore_barrier()
        pltpu.sync_copy(deg_sh.at[pl.ds(sid * SP, SP)],
                        degp_hbm.at[cid, pl.ds(sid * SP, SP)])

    return att(xn, src3, dst3, zrows)


def _msg_pass(h, src3, dst3, ew3, dinv, zrows, Dm):
    @functools.partial(
        pl.kernel,
        out_type=jax.ShapeDtypeStruct((NC, NP, Dm), jnp.float32),
        mesh=_mesh,
        compiler_params=_sc_params,
        scratch_types=[
            pltpu.VMEM((NP,), jnp.float32),
            pltpu.VMEM((NCHUNK, C), jnp.int32),
            pltpu.VMEM((NCHUNK, C), jnp.int32),
            pltpu.VMEM((NCHUNK, C), jnp.float32),
            pltpu.VMEM((C, Dm), jnp.float32),
            pltpu.VMEM((C, Dm), jnp.float32),
            pltpu.VMEM_SHARED((NP, Dm), jnp.float32),
            pltpu.SemaphoreType.DMA,
            pltpu.SemaphoreType.DMA,
            pltpu.SemaphoreType.DMA,
            pltpu.SemaphoreType.DMA,
        ],
    )
    def msg(h_hbm, src_hbm, dst_hbm, ew_hbm, dinv_hbm, z_hbm, mp_hbm,
            dinv_v, idx_sa, idx_da, ew_all, r0, r1, acc_sh,
            s0, s1, sc0, sc1):
        cid = lax.axis_index("c")
        sid = lax.axis_index("s")
        wid = sid * NC + cid
        rows_bufs, sems, scs = (r0, r1), (s0, s1), (sc0, sc1)

        pltpu.sync_copy(z_hbm, acc_sh.at[pl.ds(sid * SP, SP)])
        pltpu.sync_copy(dinv_hbm, dinv_v)
        pltpu.sync_copy(src_hbm.at[wid], idx_sa)
        pltpu.sync_copy(dst_hbm.at[wid], idx_da)
        pltpu.sync_copy(ew_hbm.at[wid], ew_all)
        plsc.subcore_barrier()

        def issue(i, p):
            pltpu.async_copy(h_hbm.at[idx_sa.at[i]], rows_bufs[p], sems[p])

        def wait_scat(p):
            pltpu.make_async_copy(
                rows_bufs[p], acc_sh.at[idx_da.at[0]], scs[p]).wait()

        def step(i, p, issue_next, wait_sc):
            if wait_sc:
                wait_scat(1 - p)
            if issue_next:
                issue(i + 1, 1 - p)
            pltpu.make_async_copy(
                h_hbm.at[idx_sa.at[i]], rows_bufs[p], sems[p]).wait()
            rows = rows_bufs[p]

            @pl.loop(0, C // L)
            def _(g):
                isv = idx_sa[i, pl.ds(g * L, L)]
                idv = idx_da[i, pl.ds(g * L, L)]
                ds_ = plsc.load_gather(dinv_v, [isv])
                dd_ = plsc.load_gather(dinv_v, [idv])
                w = ds_ * ew_all[i, pl.ds(g * L, L)] * dd_
                for e in range(L):
                    row = g * L + e
                    wv = jnp.broadcast_to(w[e], (L,))
                    for k in range(Dm // L):
                        rows[row, pl.ds(k * L, L)] = (
                            rows[row, pl.ds(k * L, L)] * wv)

            pltpu.async_copy(rows, acc_sh.at[idx_da.at[i]], scs[p], add=True)

        issue(0, 0)
        step(0, 0, True, False)

        @pl.loop(0, (NCHUNK - 3) // 2)
        def _(j):
            step(2 * j + 1, 1, True, True)
            step(2 * j + 2, 0, True, True)

        step(NCHUNK - 2, 1, True, True)
        step(NCHUNK - 1, 0, False, True)
        wait_scat(0)

        plsc.subcore_barrier()
        pltpu.sync_copy(acc_sh.at[pl.ds(sid * SP, SP)],
                        mp_hbm.at[cid, pl.ds(sid * SP, SP)])

    return msg(h, src3, dst3, ew3, dinv, zrows)


def _tc_call(body, out_shape, *args):
    return pl.pallas_call(body, out_shape=out_shape)(*args)


def _prep1(x, W1):
    def body(x_ref, w_ref, xn_ref, h1_ref):
        xv = x_ref[...]
        s = jnp.sum(xv * xv, axis=1, keepdims=True)
        na = jnp.maximum(jnp.sqrt(s), 1e-8)
        xn_ref[...] = xv / na
        h1_ref[...] = jnp.dot(xv, w_ref[...],
                              preferred_element_type=jnp.float32)
    return _tc_call(
        body,
        (jax.ShapeDtypeStruct((N, x.shape[1]), jnp.float32),
         jax.ShapeDtypeStruct((N, W1.shape[1]), jnp.float32)),
        x, W1)


def _dinv_of(degp):
    def body(degp_ref, dinv_ref):
        deg = 1.0 + jnp.sum(degp_ref[...], axis=(0, 2), keepdims=True)
        dinv_ref[...] = lax.rsqrt(deg)
    return _tc_call(body, jax.ShapeDtypeStruct((1, NP, 1), jnp.float32), degp)


def _mid(mp, h1, dinv_col, b1_row, W2):
    def body(mp_ref, h1_ref, dc_ref, b_ref, w_ref, hn_ref, h2_ref):
        dc = dc_ref[...][:N]
        h = (mp_ref[0][:N] + mp_ref[1][:N]
             + dc * dc * h1_ref[...] + b_ref[...])
        h = jnp.maximum(h, 0.0)
        s = jnp.sum(h * h, axis=1, keepdims=True)
        na = jnp.maximum(jnp.sqrt(s), 1e-8)
        hn_ref[...] = h / na
        h2_ref[...] = jnp.dot(h, w_ref[...],
                              preferred_element_type=jnp.float32)
    return _tc_call(
        body,
        (jax.ShapeDtypeStruct((N, h1.shape[1]), jnp.float32),
         jax.ShapeDtypeStruct((N, W2.shape[1]), jnp.float32)),
        mp, h1, dinv_col, b1_row, W2)


def _final(mp, h2, dinv_col, b2_row):
    def body(mp_ref, h2_ref, dc_ref, b_ref, out_ref):
        dc = dc_ref[...][:N]
        out_ref[...] = (mp_ref[0][:N] + mp_ref[1][:N]
                        + dc * dc * h2_ref[...] + b_ref[...])
    return _tc_call(
        body, jax.ShapeDtypeStruct((N, h2.shape[1]), jnp.float32),
        mp, h2, dinv_col, b2_row)


def kernel(x, adj, W1, b1, W2, b2):
    src3 = adj[0].astype(jnp.int32).reshape(NW, NCHUNK, C)
    dst3 = adj[1].astype(jnp.int32).reshape(NW, NCHUNK, C)
    z16 = jnp.zeros((SP, L), jnp.float32)

    xn, h1 = _prep1(x, W1)
    ew1, degp1 = _att_pass(xn, src3, dst3, z16, x.shape[1])
    dinv1_3 = _dinv_of(degp1)
    mp1 = _msg_pass(h1, src3, dst3, ew1, dinv1_3.reshape(NP),
                    jnp.zeros((SP, h1.shape[1]), jnp.float32), h1.shape[1])
    hn, h2 = _mid(mp1, h1, dinv1_3.reshape(NP, 1), b1.reshape(1, -1), W2)
    ew2, degp2 = _att_pass(hn, src3, dst3, z16, hn.shape[1])
    dinv2_3 = _dinv_of(degp2)
    mp2 = _msg_pass(h2, src3, dst3, ew2, dinv2_3.reshape(NP),
                    jnp.zeros((SP, h2.shape[1]), jnp.float32), h2.shape[1])
    return _final(mp2, h2, dinv2_3.reshape(NP, 1), b2.reshape(1, -1))


# speedup vs baseline: 1.4072x; 1.0675x over previous
import dataclasses
import functools

import jax
import jax.numpy as jnp
from jax import lax
from jax.experimental import pallas as pl
from jax.experimental.pallas import tpu as pltpu
from jax.experimental.pallas import tpu_sc as plsc

N = 10000
E = 320000
NC = 2
NS = 16
NW = NC * NS
ET = E // NW
C = 80
NCHUNK = ET // C
L = 16
NP = 10112
SP = NP // NS
ETP = ET + 2 * C
NCHP = ETP // C

_mesh = plsc.VectorSubcoreMesh(core_axis_name="c", subcore_axis_name="s")

_sc_params = pltpu.CompilerParams()
if "needs_layout_passes" in pltpu.CompilerParams.__dataclass_fields__:
    _sc_params = dataclasses.replace(
        _sc_params, needs_layout_passes=False, use_tc_tiling_on_sc=False)


def _att_pass(xn, src3, dst3, zrows, D):
    @functools.partial(
        pl.kernel,
        out_type=(
            jax.ShapeDtypeStruct((NC, NP, L), jnp.float32),
            jax.ShapeDtypeStruct((NW, ETP), jnp.int32),
            jax.ShapeDtypeStruct((NW, ETP), jnp.int32),
            jax.ShapeDtypeStruct((NW, ETP), jnp.float32),
            jax.ShapeDtypeStruct((NW, L), jnp.int32),
        ),
        mesh=_mesh,
        compiler_params=_sc_params,
        scratch_types=[
            pltpu.VMEM((NCHUNK, C), jnp.int32),
            pltpu.VMEM((NCHUNK, C), jnp.int32),
            pltpu.VMEM((C, D), jnp.float32),
            pltpu.VMEM((C, D), jnp.float32),
            pltpu.VMEM((C, D), jnp.float32),
            pltpu.VMEM((C, D), jnp.float32),
            pltpu.VMEM((NCHUNK, C), jnp.float32),
            pltpu.VMEM((C, L), jnp.float32),
            pltpu.VMEM((C, L), jnp.float32),
            pltpu.VMEM((ETP,), jnp.int32),
            pltpu.VMEM((ETP,), jnp.int32),
            pltpu.VMEM((ETP,), jnp.float32),
            pltpu.VMEM((L,), jnp.int32),
            pltpu.SMEM((1,), jnp.int32),
            pltpu.VMEM_SHARED((NP, L), jnp.float32),
            pltpu.SemaphoreType.DMA,
            pltpu.SemaphoreType.DMA,
            pltpu.SemaphoreType.DMA,
            pltpu.SemaphoreType.DMA,
            pltpu.SemaphoreType.DMA,
            pltpu.SemaphoreType.DMA,
        ],
    )
    def att(xn_hbm, src_hbm, dst_hbm, z_hbm,
            degp_hbm, cs_hbm, cd_hbm, cw_hbm, cnt_hbm,
            idx_sa, idx_da, a0, a1, b0, b1, ew_all, dr0, dr1,
            csrc, cdst, cew, cntv, off_sm, deg_sh,
            sa0, sa1, sb0, sb1, sd0, sd1):
        cid = lax.axis_index("c")
        sid = lax.axis_index("s")
        wid = sid * NC + cid
        a_bufs, b_bufs = (a0, a1), (b0, b1)
        drows = (dr0, dr1)
        sas, sbs, sds = (sa0, sa1), (sb0, sb1), (sd0, sd1)

        pltpu.sync_copy(z_hbm.at[pl.ds(0, SP)], deg_sh.at[pl.ds(sid * SP, SP)])
        pltpu.sync_copy(z_hbm.at[pl.ds(0, C)], dr0)
        pltpu.sync_copy(z_hbm.at[pl.ds(0, C)], dr1)
        pltpu.sync_copy(src_hbm.at[wid], idx_sa)
        pltpu.sync_copy(dst_hbm.at[wid], idx_da)
        plsc.subcore_barrier()
        off_sm[0] = 0

        lane_iota = lax.iota(jnp.int32, L)
        zeros_i = jnp.zeros((L,), jnp.int32)
        zeros_f = jnp.zeros((L,), jnp.float32)

        @pl.loop(0, ETP // L)
        def _(t):
            cew[pl.ds(t * L, L)] = zeros_f
            csrc[pl.ds(t * L, L)] = zeros_i
            cdst[pl.ds(t * L, L)] = zeros_i

        def issue(i, p):
            pltpu.async_copy(xn_hbm.at[idx_sa.at[i]], a_bufs[p], sas[p])
            pltpu.async_copy(xn_hbm.at[idx_da.at[i]], b_bufs[p], sbs[p])

        def wait_dscat(p):
            pltpu.make_async_copy(
                drows[p], deg_sh.at[idx_da.at[0]], sds[p]).wait()

        def step(i, p, issue_next, wait_d):
            if issue_next:
                issue(i + 1, 1 - p)
            pltpu.make_async_copy(
                xn_hbm.at[idx_sa.at[i]], a_bufs[p], sas[p]).wait()
            pltpu.make_async_copy(
                xn_hbm.at[idx_da.at[i]], b_bufs[p], sbs[p]).wait()
            if wait_d:
                wait_dscat(p)
            a_buf, b_buf, drow = a_bufs[p], b_bufs[p], drows[p]

            @pl.loop(0, C // 4)
            def _(q):
                sims4 = jnp.zeros((L,), jnp.float32)
                for e in range(4):
                    row = q * 4 + e
                    acc = a_buf[row, pl.ds(0, L)] * b_buf[row, pl.ds(0, L)]
                    for k in range(1, D // L):
                        acc += (a_buf[row, pl.ds(k * L, L)]
                                * b_buf[row, pl.ds(k * L, L)])
                    sims4 = jnp.where(lane_iota == e,
                                      jnp.broadcast_to(jnp.sum(acc), (L,)),
                                      sims4)
                sims4 = jnp.where(sims4 < 0.1, 0.0, sims4)
                plsc.store_scatter(
                    ew_all,
                    [jnp.full((L,), i, jnp.int32), q * 4 + lane_iota],
                    sims4, mask=lane_iota < 4)

            @pl.loop(0, C // L)
            def _(g):
                s16 = ew_all[i, pl.ds(g * L, L)]
                plsc.store_scatter(drow, [g * L + lane_iota, zeros_i], s16)
                mask = s16 > 0.0
                off = off_sm[0]
                cs = plsc.cumsum(mask.astype(jnp.int32))
                pos = off + cs - 1
                plsc.store_scatter(cew, [pos], s16, mask=mask)
                plsc.store_scatter(csrc, [pos], idx_sa[i, pl.ds(g * L, L)],
                                   mask=mask)
                plsc.store_scatter(cdst, [pos], idx_da[i, pl.ds(g * L, L)],
                                   mask=mask)
                off_sm[0] = off + cs[L - 1]

            pltpu.async_copy(drow, deg_sh.at[idx_da.at[i]], sds[p], add=True)

        issue(0, 0)
        step(0, 0, True, False)
        step(1, 1, True, False)

        @pl.loop(0, (NCHUNK - 3) // 2)
        def _(j):
            step(2 * j + 2, 0, True, True)
            step(2 * j + 3, 1, True, True)

        step(NCHUNK - 1, 0, False, True)
        wait_dscat(1)
        wait_dscat(0)

        cntv[...] = jnp.full((L,), off_sm[0], jnp.int32)
        pltpu.sync_copy(csrc, cs_hbm.at[wid])
        pltpu.sync_copy(cdst, cd_hbm.at[wid])
        pltpu.sync_copy(cew, cw_hbm.at[wid])
        pltpu.sync_copy(cntv, cnt_hbm.at[wid])
        plsc.subcore_barrier()
        pltpu.sync_copy(deg_sh.at[pl.ds(sid * SP, SP)],
                        degp_hbm.at[cid, pl.ds(sid * SP, SP)])

    return att(xn, src3, dst3, zrows)


def _msg_pass(h, src3, dst3, ew3, cnts, dinv, zrows, Dm):
    @functools.partial(
        pl.kernel,
        out_type=jax.ShapeDtypeStruct((NC, NP, Dm), jnp.float32),
        mesh=_mesh,
        compiler_params=_sc_params,
        scratch_types=[
            pltpu.VMEM((NP,), jnp.float32),
            pltpu.VMEM((NCHP, C), jnp.int32),
            pltpu.VMEM((NCHP, C), jnp.int32),
            pltpu.VMEM((NCHP, C), jnp.float32),
            pltpu.VMEM((L,), jnp.int32),
            pltpu.VMEM((C, Dm), jnp.float32),
            pltpu.VMEM((C, Dm), jnp.float32),
            pltpu.VMEM_SHARED((NP, Dm), jnp.float32),
            pltpu.SemaphoreType.DMA,
            pltpu.SemaphoreType.DMA,
            pltpu.SemaphoreType.DMA,
            pltpu.SemaphoreType.DMA,
        ],
    )
    def msg(h_hbm, src_hbm, dst_hbm, ew_hbm, cnt_hbm, dinv_hbm, z_hbm,
            mp_hbm, dinv_v, idx_sa, idx_da, ew_all, cntb, r0, r1, acc_sh,
            s0, s1, sc0, sc1):
        cid = lax.axis_index("c")
        sid = lax.axis_index("s")
        wid = sid * NC + cid
        rows_bufs, sems, scs = (r0, r1), (s0, s1), (sc0, sc1)

        pltpu.sync_copy(z_hbm, acc_sh.at[pl.ds(sid * SP, SP)])
        pltpu.sync_copy(dinv_hbm, dinv_v)
        pltpu.sync_copy(src_hbm.at[wid], idx_sa)
        pltpu.sync_copy(dst_hbm.at[wid], idx_da)
        pltpu.sync_copy(ew_hbm.at[wid], ew_all)
        pltpu.sync_copy(cnt_hbm.at[wid], cntb)
        plsc.subcore_barrier()
        cnt = cntb[pl.ds(0, L)][0]
        nch = lax.div(cnt + (C - 1), C)

        def issue(i, p):
            pltpu.async_copy(h_hbm.at[idx_sa.at[i]], rows_bufs[p], sems[p])

        def wait_scat(p):
            pltpu.make_async_copy(
                rows_bufs[p], acc_sh.at[idx_da.at[0]], scs[p]).wait()

        def step(i, p, wait_sc):
            @pl.when(i < nch)
            def _():
                if wait_sc:
                    wait_scat(1 - p)

                @pl.when(i + 1 < nch)
                def _():
                    issue(i + 1, 1 - p)

                pltpu.make_async_copy(
                    h_hbm.at[idx_sa.at[i]], rows_bufs[p], sems[p]).wait()
                rows = rows_bufs[p]

                @pl.loop(0, C // L)
                def _(g):
                    isv = idx_sa[i, pl.ds(g * L, L)]
                    idv = idx_da[i, pl.ds(g * L, L)]
                    ds_ = plsc.load_gather(dinv_v, [isv])
                    dd_ = plsc.load_gather(dinv_v, [idv])
                    w = ds_ * ew_all[i, pl.ds(g * L, L)] * dd_
                    for e in range(L):
                        row = g * L + e
                        wv = jnp.broadcast_to(w[e], (L,))
                        for k in range(Dm // L):
                            rows[row, pl.ds(k * L, L)] = (
                                rows[row, pl.ds(k * L, L)] * wv)

                pltpu.async_copy(rows, acc_sh.at[idx_da.at[i]], scs[p],
                                 add=True)

        @pl.when(nch > 0)
        def _():
            issue(0, 0)
        step(0, 0, False)

        @pl.loop(0, (NCHP - 1) // 2)
        def _(j):
            step(2 * j + 1, 1, True)
            step(2 * j + 2, 0, True)

        @pl.when((nch > 0) & (lax.rem(nch - 1, 2) == 0))
        def _():
            wait_scat(0)

        @pl.when((nch > 0) & (lax.rem(nch - 1, 2) == 1))
        def _():
            wait_scat(1)

        plsc.subcore_barrier()
        pltpu.sync_copy(acc_sh.at[pl.ds(sid * SP, SP)],
                        mp_hbm.at[cid, pl.ds(sid * SP, SP)])

    return msg(h, src3, dst3, ew3, cnts, dinv, zrows)


def _tc_call(body, out_shape, *args):
    return pl.pallas_call(body, out_shape=out_shape)(*args)


def _prep1(x, W1):
    def body(x_ref, w_ref, xn_ref, h1_ref):
        xv = x_ref[...]
        s = jnp.sum(xv * xv, axis=1, keepdims=True)
        na = jnp.maximum(jnp.sqrt(s), 1e-8)
        xn_ref[...] = xv / na
        h1_ref[...] = jnp.dot(xv, w_ref[...],
                              preferred_element_type=jnp.float32)
    return _tc_call(
        body,
        (jax.ShapeDtypeStruct((N, x.shape[1]), jnp.float32),
         jax.ShapeDtypeStruct((N, W1.shape[1]), jnp.float32)),
        x, W1)


def _dinv_of(degp):
    def body(degp_ref, dinv_ref):
        deg = 1.0 + jnp.sum(degp_ref[...], axis=(0, 2), keepdims=True)
        dinv_ref[...] = lax.rsqrt(deg)
    return _tc_call(body, jax.ShapeDtypeStruct((1, NP, 1), jnp.float32), degp)


def _mid(mp, h1, dinv_col, b1_row, W2):
    def body(mp_ref, h1_ref, dc_ref, b_ref, w_ref, hn_ref, h2_ref):
        dc = dc_ref[...][:N]
        h = (mp_ref[0][:N] + mp_ref[1][:N]
             + dc * dc * h1_ref[...] + b_ref[...])
        h = jnp.maximum(h, 0.0)
        s = jnp.sum(h * h, axis=1, keepdims=True)
        na = jnp.maximum(jnp.sqrt(s), 1e-8)
        hn_ref[...] = h / na
        h2_ref[...] = jnp.dot(h, w_ref[...],
                              preferred_element_type=jnp.float32)
    return _tc_call(
        body,
        (jax.ShapeDtypeStruct((N, h1.shape[1]), jnp.float32),
         jax.ShapeDtypeStruct((N, W2.shape[1]), jnp.float32)),
        mp, h1, dinv_col, b1_row, W2)


def _final(mp, h2, dinv_col, b2_row):
    def body(mp_ref, h2_ref, dc_ref, b_ref, out_ref):
        dc = dc_ref[...][:N]
        out_ref[...] = (mp_ref[0][:N] + mp_ref[1][:N]
                        + dc * dc * h2_ref[...] + b_ref[...])
    return _tc_call(
        body, jax.ShapeDtypeStruct((N, h2.shape[1]), jnp.float32),
        mp, h2, dinv_col, b2_row)


def kernel(x, adj, W1, b1, W2, b2):
    src3 = adj[0].astype(jnp.int32).reshape(NW, NCHUNK, C)
    dst3 = adj[1].astype(jnp.int32).reshape(NW, NCHUNK, C)
    z16 = jnp.zeros((SP, L), jnp.float32)

    xn, h1 = _prep1(x, W1)
    degp1, cs1, cd1, cw1, cnt1 = _att_pass(xn, src3, dst3, z16, x.shape[1])
    dinv1_3 = _dinv_of(degp1)
    mp1 = _msg_pass(h1, cs1.reshape(NW, NCHP, C), cd1.reshape(NW, NCHP, C),
                    cw1.reshape(NW, NCHP, C), cnt1, dinv1_3.reshape(NP),
                    jnp.zeros((SP, h1.shape[1]), jnp.float32), h1.shape[1])
    hn, h2 = _mid(mp1, h1, dinv1_3.reshape(NP, 1), b1.reshape(1, -1), W2)
    degp2, cs2, cd2, cw2, cnt2 = _att_pass(hn, src3, dst3, z16, hn.shape[1])
    dinv2_3 = _dinv_of(degp2)
    mp2 = _msg_pass(h2, cs2.reshape(NW, NCHP, C), cd2.reshape(NW, NCHP, C),
                    cw2.reshape(NW, NCHP, C), cnt2, dinv2_3.reshape(NP),
                    jnp.zeros((SP, h2.shape[1]), jnp.float32), h2.shape[1])
    return _final(mp2, h2, dinv2_3.reshape(NP, 1), b2.reshape(1, -1))
